# Initial kernel scaffold; baseline (speedup 1.0000x reference)
import jax
import jax.numpy as jnp
from jax.experimental import pallas as pl


def kernel(x, edge_index, edge_attr, global_attr, W1, b1, W2, b2, gamma, beta):
    raise NotImplementedError("write your pallas kernel here")


# speedup vs baseline: 3.6020x; 3.6020x over previous
import functools

import jax
import jax.numpy as jnp
from jax import lax
from jax.experimental import pallas as pl
from jax.experimental.pallas import tpu as pltpu
from jax.experimental.pallas import tpu_sc as plsc

_NC = 2
_NS = 16
_NW = _NC * _NS


def _precompute_body(x_ref, w1_ref, g_ref, b1_ref, t_ref, c0_ref):
    h = x_ref.shape[-1]
    xv = x_ref[...]
    t_ref[0, :, :] = jnp.dot(xv, w1_ref[0:h, :], preferred_element_type=jnp.float32)
    t_ref[1, :, :] = jnp.dot(xv, w1_ref[h:2 * h, :], preferred_element_type=jnp.float32)
    c0_ref[...] = (
        jnp.dot(g_ref[...], w1_ref[3 * h:4 * h, :], preferred_element_type=jnp.float32)
        + b1_ref[...]
    )


@functools.lru_cache(maxsize=None)
def _make_gather(e2, h, chunk):
    per_w = e2 // _NW
    steps = per_w // chunk
    mesh = plsc.VectorSubcoreMesh(core_axis_name="c", subcore_axis_name="s")

    @functools.partial(
        pl.kernel,
        mesh=mesh,
        out_type=jax.ShapeDtypeStruct((e2, h), jnp.float32),
        scratch_types=[
            pltpu.VMEM((chunk,), jnp.int32),
            pltpu.VMEM((chunk, h), jnp.float32),
            pltpu.SemaphoreType.DMA,
        ],
    )
    def gather_kernel(t_hbm, j_hbm, out_hbm, idx_v, rows_v, sem):
        wid = lax.axis_index("s") * _NC + lax.axis_index("c")
        base = wid * per_w

        def body(i, carry):
            off = base + i * chunk
            pltpu.sync_copy(j_hbm.at[pl.ds(off, chunk)], idx_v)
            pltpu.async_copy(t_hbm.at[idx_v], rows_v, sem).wait()
            pltpu.sync_copy(rows_v, out_hbm.at[pl.ds(off, chunk)])
            return carry

        lax.fori_loop(0, steps, body, 0)

    return gather_kernel


def _mlp_body(gt_ref, gs_ref, ea_ref, c_ref, c0_ref, w2_ref, b2_ref,
              gamma_ref, beta_ref, out_ref):
    pre = (
        gt_ref[0]
        + gs_ref[0]
        + jnp.dot(ea_ref[...], c_ref[...], preferred_element_type=jnp.float32)
        + c0_ref[...]
    )
    h1 = jnp.maximum(pre, 0.0)
    h2 = jnp.dot(h1, w2_ref[...], preferred_element_type=jnp.float32) + b2_ref[...]
    mean = jnp.mean(h2, axis=-1, keepdims=True)
    d = h2 - mean
    var = jnp.mean(d * d, axis=-1, keepdims=True)
    out_ref[...] = d * lax.rsqrt(var + 1e-5) * gamma_ref[...] + beta_ref[...]


def kernel(x, edge_index, edge_attr, global_attr, W1, b1, W2, b2, gamma, beta):
    n, h = x.shape
    e = edge_attr.shape[0]

    tbl3, c0 = pl.pallas_call(
        _precompute_body,
        out_shape=[
            jax.ShapeDtypeStruct((2, n, h), jnp.float32),
            jax.ShapeDtypeStruct((1, h), jnp.float32),
        ],
    )(x, W1, global_attr.reshape(1, h), b1.reshape(1, h))
    tbl = tbl3.reshape(2 * n, h)

    j = jnp.concatenate([edge_index[1], edge_index[0] + n]).astype(jnp.int32)

    chunk = 400
    g = _make_gather(2 * e, h, chunk)(tbl, j)
    g3 = g.reshape(2, e, h)

    be = 2000
    grid = (e // be,)
    out = pl.pallas_call(
        _mlp_body,
        grid=grid,
        in_specs=[
            pl.BlockSpec((1, be, h), lambda i: (0, i, 0)),
            pl.BlockSpec((1, be, h), lambda i: (1, i, 0)),
            pl.BlockSpec((be, h), lambda i: (i, 0)),
            pl.BlockSpec((h, h), lambda i: (0, 0)),
            pl.BlockSpec((1, h), lambda i: (0, 0)),
            pl.BlockSpec((h, h), lambda i: (0, 0)),
            pl.BlockSpec((1, h), lambda i: (0, 0)),
            pl.BlockSpec((1, h), lambda i: (0, 0)),
            pl.BlockSpec((1, h), lambda i: (0, 0)),
        ],
        out_specs=pl.BlockSpec((be, h), lambda i: (i, 0)),
        out_shape=jax.ShapeDtypeStruct((e, h), jnp.float32),
    )(g3, g3, edge_attr, W1[2 * h:3 * h, :], c0, W2, b2.reshape(1, h),
      gamma.reshape(1, h), beta.reshape(1, h))
    return out


# speedup vs baseline: 4.1529x; 1.1530x over previous
import functools

import jax
import jax.numpy as jnp
from jax import lax
from jax.experimental import pallas as pl
from jax.experimental.pallas import tpu as pltpu
from jax.experimental.pallas import tpu_sc as plsc

_NC = 2
_NS = 16
_NW = _NC * _NS


def _precompute_body(x_ref, w1_ref, g_ref, b1_ref, t_ref, c0_ref):
    h = x_ref.shape[-1]
    xv = x_ref[...]
    t_ref[0, :, :] = jnp.dot(xv, w1_ref[0:h, :], preferred_element_type=jnp.float32)
    t_ref[1, :, :] = jnp.dot(xv, w1_ref[h:2 * h, :], preferred_element_type=jnp.float32)
    c0_ref[...] = (
        jnp.dot(g_ref[...], w1_ref[3 * h:4 * h, :], preferred_element_type=jnp.float32)
        + b1_ref[...]
    )


@functools.lru_cache(maxsize=None)
def _make_gather(e2, h, chunk, dtype=jnp.float32):
    per_w = e2 // _NW
    steps = per_w // chunk
    mesh = plsc.VectorSubcoreMesh(core_axis_name="c", subcore_axis_name="s")

    @functools.partial(
        pl.kernel,
        mesh=mesh,
        out_type=jax.ShapeDtypeStruct((e2, h), dtype),
        scratch_types=[
            pltpu.VMEM((chunk,), jnp.int32),
            pltpu.VMEM((chunk, h), dtype),
            pltpu.SemaphoreType.DMA,
        ],
    )
    def gather_kernel(t_hbm, j_hbm, out_hbm, idx_v, rows_v, sem):
        wid = lax.axis_index("s") * _NC + lax.axis_index("c")
        base = wid * per_w

        def body(i, carry):
            off = base + i * chunk
            pltpu.sync_copy(j_hbm.at[pl.ds(off, chunk)], idx_v)
            pltpu.async_copy(t_hbm.at[idx_v], rows_v, sem).wait()
            pltpu.sync_copy(rows_v, out_hbm.at[pl.ds(off, chunk)])
            return carry

        lax.fori_loop(0, steps, body, 0)

    return gather_kernel


def _mlp_body(gt_ref, gs_ref, ea_ref, c_ref, c0_ref, w2_ref, b2_ref,
              gamma_ref, beta_ref, out_ref):
    pre = (
        gt_ref[0].astype(jnp.float32)
        + gs_ref[0].astype(jnp.float32)
        + jnp.dot(ea_ref[...], c_ref[...], preferred_element_type=jnp.float32)
        + c0_ref[...]
    )
    h1 = jnp.maximum(pre, 0.0)
    h2 = jnp.dot(h1, w2_ref[...], preferred_element_type=jnp.float32) + b2_ref[...]
    mean = jnp.mean(h2, axis=-1, keepdims=True)
    d = h2 - mean
    var = jnp.mean(d * d, axis=-1, keepdims=True)
    out_ref[...] = d * lax.rsqrt(var + 1e-5) * gamma_ref[...] + beta_ref[...]


def _mlp_body_aliased(acc_ref, gt_ref, gs_ref, ea_ref, c_ref, c0_ref, w2_ref,
                      b2_ref, gamma_ref, beta_ref, out_ref):
    del acc_ref
    _mlp_body(gt_ref, gs_ref, ea_ref, c_ref, c0_ref, w2_ref, b2_ref,
              gamma_ref, beta_ref, out_ref)


def kernel(x, edge_index, edge_attr, global_attr, W1, b1, W2, b2, gamma, beta):
    n, h = x.shape
    e = edge_attr.shape[0]

    tbl3, c0 = pl.pallas_call(
        _precompute_body,
        out_shape=[
            jax.ShapeDtypeStruct((2, n, h), jnp.float32),
            jax.ShapeDtypeStruct((1, h), jnp.float32),
        ],
    )(x, W1, global_attr.reshape(1, h), b1.reshape(1, h))
    tbl = tbl3.reshape(2 * n, h)

    ns = 5
    es = e // ns
    chunk = 400
    be = 2000
    bps = es // be

    tgt = edge_index[1].astype(jnp.int32).reshape(ns, es)
    src = (edge_index[0].astype(jnp.int32) + n).reshape(ns, es)
    gather = _make_gather(2 * es, h, chunk)
    gs_list = [
        gather(tbl, jnp.concatenate([tgt[k], src[k]])).reshape(2, es, h)
        for k in range(ns)
    ]

    weights = (W1[2 * h:3 * h, :], c0, W2, b2.reshape(1, h),
               gamma.reshape(1, h), beta.reshape(1, h))
    w_specs = [
        pl.BlockSpec((h, h), lambda i: (0, 0)),
        pl.BlockSpec((1, h), lambda i: (0, 0)),
        pl.BlockSpec((h, h), lambda i: (0, 0)),
        pl.BlockSpec((1, h), lambda i: (0, 0)),
        pl.BlockSpec((1, h), lambda i: (0, 0)),
        pl.BlockSpec((1, h), lambda i: (0, 0)),
    ]

    def stripe_specs(k):
        off = k * bps
        return [
            pl.BlockSpec((1, be, h), lambda i: (0, i, 0)),
            pl.BlockSpec((1, be, h), lambda i: (1, i, 0)),
            pl.BlockSpec((be, h), lambda i: (off + i, 0)),
        ] + w_specs

    out_sds = jax.ShapeDtypeStruct((e, h), jnp.float32)

    out = pl.pallas_call(
        _mlp_body,
        grid=(bps,),
        in_specs=stripe_specs(0),
        out_specs=pl.BlockSpec((be, h), lambda i: (i, 0)),
        out_shape=out_sds,
    )(gs_list[0], gs_list[0], edge_attr, *weights)

    for k in range(1, ns):
        off = k * bps
        out = pl.pallas_call(
            _mlp_body_aliased,
            grid=(bps,),
            in_specs=[pl.BlockSpec(memory_space=pltpu.MemorySpace.HBM)]
            + stripe_specs(k),
            out_specs=pl.BlockSpec((be, h),
                                   lambda i, off=off: (off + i, 0)),
            out_shape=out_sds,
            input_output_aliases={0: 0},
        )(out, gs_list[k], gs_list[k], edge_attr, *weights)
    return out


# speedup vs baseline: 5.0022x; 1.2045x over previous
import functools

import jax
import jax.numpy as jnp
from jax import lax
from jax.experimental import pallas as pl
from jax.experimental.pallas import tpu as pltpu
from jax.experimental.pallas import tpu_sc as plsc

_NC = 2
_NS = 16
_NW = _NC * _NS


def _pack_bf16(v):
    k = v.shape[-1] // 2
    ua = lax.bitcast_convert_type(v[:, :k], jnp.uint32)
    ub = lax.bitcast_convert_type(v[:, k:], jnp.uint32)
    hi = (ua + jnp.uint32(0x8000)) & jnp.uint32(0xFFFF0000)
    lo = (ub + jnp.uint32(0x8000)) >> jnp.uint32(16)
    return lax.bitcast_convert_type(hi | lo, jnp.float32)


def _unpack_bf16(w):
    u = lax.bitcast_convert_type(w, jnp.uint32)
    a = lax.bitcast_convert_type(u & jnp.uint32(0xFFFF0000), jnp.float32)
    b = lax.bitcast_convert_type(u << jnp.uint32(16), jnp.float32)
    return jnp.concatenate([a, b], axis=-1)


def _precompute_body(x_ref, w1_ref, g_ref, b1_ref, t_ref, c0_ref):
    h = x_ref.shape[-1]
    xv = x_ref[...]
    t_ref[0, :, :] = jnp.dot(xv, w1_ref[0:h, :], preferred_element_type=jnp.float32)
    t_ref[1, :, :] = jnp.dot(xv, w1_ref[h:2 * h, :], preferred_element_type=jnp.float32)
    c0_ref[...] = (
        jnp.dot(g_ref[...], w1_ref[3 * h:4 * h, :], preferred_element_type=jnp.float32)
        + b1_ref[...]
    )


@functools.lru_cache(maxsize=None)
def _make_gather(es, h, chunk):
    per_w = es // _NW
    steps = per_w // chunk
    mesh = plsc.VectorSubcoreMesh(core_axis_name="c", subcore_axis_name="s")

    @functools.partial(
        pl.kernel,
        mesh=mesh,
        out_type=jax.ShapeDtypeStruct((es, h), jnp.float32),
        scratch_types=[
            pltpu.VMEM((chunk,), jnp.int32),
            pltpu.VMEM((chunk,), jnp.int32),
            pltpu.VMEM((chunk, h), jnp.float32),
            pltpu.SemaphoreType.DMA,
        ],
    )
    def gather_kernel(t_hbm, tgt_hbm, src_hbm, out_hbm, idx_t, idx_s, rows_v, sem):
        wid = lax.axis_index("s") * _NC + lax.axis_index("c")
        base = wid * per_w

        def body(i, carry):
            off = base + i * chunk
            pltpu.sync_copy(tgt_hbm.at[pl.ds(off, chunk)], idx_t)
            pltpu.sync_copy(src_hbm.at[pl.ds(off, chunk)], idx_s)
            pltpu.async_copy(t_hbm.at[idx_t], rows_v, sem).wait()
            pltpu.async_copy(t_hbm.at[idx_s], rows_v, sem, add=True).wait()
            pltpu.sync_copy(rows_v, out_hbm.at[pl.ds(off, chunk)])
            return carry

        lax.fori_loop(0, steps, body, 0)

    return gather_kernel


def _mlp_body(g_ref, ea_ref, c_ref, c0_ref, w2_ref, b2_ref,
              gamma_ref, beta_ref, out_ref):
    pre = (
        g_ref[...]
        + jnp.dot(ea_ref[...], c_ref[...], preferred_element_type=jnp.float32)
        + c0_ref[...]
    )
    h1 = jnp.maximum(pre, 0.0)
    h2v = jnp.dot(h1, w2_ref[...], preferred_element_type=jnp.float32) + b2_ref[...]
    mean = jnp.mean(h2v, axis=-1, keepdims=True)
    d = h2v - mean
    var = jnp.mean(d * d, axis=-1, keepdims=True)
    out_ref[...] = d * lax.rsqrt(var + 1e-5) * gamma_ref[...] + beta_ref[...]


def _mlp_body_aliased(acc_ref, g_ref, ea_ref, c_ref, c0_ref, w2_ref,
                      b2_ref, gamma_ref, beta_ref, out_ref):
    del acc_ref
    _mlp_body(g_ref, ea_ref, c_ref, c0_ref, w2_ref, b2_ref,
              gamma_ref, beta_ref, out_ref)


def kernel(x, edge_index, edge_attr, global_attr, W1, b1, W2, b2, gamma, beta):
    n, h = x.shape
    e = edge_attr.shape[0]

    tbl3, c0 = pl.pallas_call(
        _precompute_body,
        out_shape=[
            jax.ShapeDtypeStruct((2, n, h), jnp.float32),
            jax.ShapeDtypeStruct((1, h), jnp.float32),
        ],
    )(x, W1, global_attr.reshape(1, h), b1.reshape(1, h))
    tbl = tbl3.reshape(2 * n, h)

    ns = 5
    es = e // ns
    chunk = 400
    be = 2000
    bps = es // be

    tgt = edge_index[1].astype(jnp.int32).reshape(ns, es)
    src = (edge_index[0].astype(jnp.int32) + n).reshape(ns, es)
    gather = _make_gather(es, h, chunk)
    g_list = [gather(tbl, tgt[k], src[k]) for k in range(ns)]

    weights = (W1[2 * h:3 * h, :], c0, W2, b2.reshape(1, h),
               gamma.reshape(1, h), beta.reshape(1, h))
    w_specs = [
        pl.BlockSpec((h, h), lambda i: (0, 0)),
        pl.BlockSpec((1, h), lambda i: (0, 0)),
        pl.BlockSpec((h, h), lambda i: (0, 0)),
        pl.BlockSpec((1, h), lambda i: (0, 0)),
        pl.BlockSpec((1, h), lambda i: (0, 0)),
        pl.BlockSpec((1, h), lambda i: (0, 0)),
    ]

    def stripe_specs(k):
        off = k * bps
        return [
            pl.BlockSpec((be, h), lambda i: (i, 0)),
            pl.BlockSpec((be, h), lambda i, off=off: (off + i, 0)),
        ] + w_specs

    out_sds = jax.ShapeDtypeStruct((e, h), jnp.float32)

    out = pl.pallas_call(
        _mlp_body,
        grid=(bps,),
        in_specs=stripe_specs(0),
        out_specs=pl.BlockSpec((be, h), lambda i: (i, 0)),
        out_shape=out_sds,
    )(g_list[0], edge_attr, *weights)

    for k in range(1, ns):
        off = k * bps
        out = pl.pallas_call(
            _mlp_body_aliased,
            grid=(bps,),
            in_specs=[pl.BlockSpec(memory_space=pltpu.MemorySpace.HBM)]
            + stripe_specs(k),
            out_specs=pl.BlockSpec((be, h),
                                   lambda i, off=off: (off + i, 0)),
            out_shape=out_sds,
            input_output_aliases={0: 0},
        )(out, g_list[k], edge_attr, *weights)
    return out
